# baseline (speedup 1.0000x reference)
import functools

import jax
import jax.numpy as jnp
from jax import lax
from jax.experimental import pallas as pl
from jax.experimental.pallas import tpu as pltpu
from jax.experimental.pallas import tpu_sc as plsc

_B = 32
_T = 64
_IN_CH = 263
_CH = 256
_ZCH = 512
_NB = 768

_BF = jnp.bfloat16


def _mm(a, b):
    return lax.dot_general(a.astype(_BF), b.astype(_BF),
                           (((1,), (0,)), ((), ())),
                           preferred_element_type=jnp.float32)


def _mm3(h, w):
    bb, tt, ci = h.shape
    return _mm(h.reshape(bb * tt, ci), w).reshape(bb, tt, w.shape[1])


def _shift_fwd(h):
    return jnp.concatenate([jnp.zeros_like(h[:, :1]), h[:, :-1]], axis=1)


def _shift_bwd(h):
    return jnp.concatenate([h[:, 1:], jnp.zeros_like(h[:, :1])], axis=1)


def _conv3(h, wt, b):
    y = _mm3(_shift_fwd(h), wt[0])
    y = y + _mm3(h, wt[1])
    y = y + _mm3(_shift_bwd(h), wt[2])
    return y + b[None, None, :]


def _resblock(h, w1, b1, w2, b2, ws=None, bs=None):
    u = _conv3(jnp.maximum(h, 0.0), w1, b1)
    u = _conv3(jnp.maximum(u, 0.0), w2, b2)
    if ws is not None:
        h = _mm3(h, ws) + bs[None, None, :]
    return h + u


def _downconv(he, ho, wt, b):
    y = _mm3(he, wt[0]) + _mm3(ho, wt[1]) + _mm3(_shift_bwd(he), wt[2])
    return y + b[None, None, :]


def _upconv(h, wt, b):
    ye = (_mm3(_shift_fwd(h), wt[0]) + _mm3(h, wt[1]) + _mm3(h, wt[2])
          + b[None, None, :])
    yo = (_mm3(h, wt[0]) + _mm3(h, wt[1]) + _mm3(_shift_bwd(h), wt[2])
          + b[None, None, :])
    return ye, yo


def _call(body, args, out_shapes):
    return pl.pallas_call(
        body,
        out_shape=[jax.ShapeDtypeStruct(s, d) for (s, d) in out_shapes],
    )(*args)


def _enc0_body(x, w_in, b_in, w1a, b1a, w2a, b2a, w1b, b1b, w2b, b2b, out):
    h = _conv3(x[...], w_in[...], b_in[...])
    h = _resblock(h, w1a[...], b1a[...], w2a[...], b2a[...])
    h = _resblock(h, w1b[...], b1b[...], w2b[...], b2b[...])
    out[...] = h


def _enc1_body(he, ho, wd, bd, w1a, b1a, w2a, b2a, wsa, bsa,
               w1b, b1b, w2b, b2b, out):
    h = _downconv(he[...], ho[...], wd[...], bd[...])
    h = _resblock(h, w1a[...], b1a[...], w2a[...], b2a[...], wsa[...], bsa[...])
    h = _resblock(h, w1b[...], b1b[...], w2b[...], b2b[...])
    out[...] = h


def _enc2_body(he, ho, wd, bd, w1a, b1a, w2a, b2a, wsa, bsa,
               w1b, b1b, w2b, b2b, wo, bo, cbt, idx_out):
    h = _downconv(he[...], ho[...], wd[...], bd[...])
    h = _resblock(h, w1a[...], b1a[...], w2a[...], b2a[...], wsa[...], bsa[...])
    h = _resblock(h, w1b[...], b1b[...], w2b[...], b2b[...])
    z = _conv3(jnp.maximum(h, 0.0), wo[...], bo[...])
    bb, tt, ci = z.shape
    zf = z.reshape(bb * tt, ci)
    cbt_v = cbt[...]
    s = _mm(zf, cbt_v)
    cn = jnp.sum(cbt_v * cbt_v, axis=0)
    d = cn[None, :] - 2.0 * s
    m = jnp.min(d, axis=1, keepdims=True)
    cols = lax.broadcasted_iota(jnp.int32, d.shape, 1)
    idx = jnp.min(jnp.where(d == m, cols, _NB), axis=1)
    idx_out[...] = idx.astype(jnp.int32)


def _dec2_body(zq, w_in, b_in, w1a, b1a, w2a, b2a, w1b, b1b, w2b, b2b,
               wu, bu, oute, outo):
    h = _conv3(zq[...], w_in[...], b_in[...])
    h = _resblock(h, w1a[...], b1a[...], w2a[...], b2a[...])
    h = _resblock(h, w1b[...], b1b[...], w2b[...], b2b[...])
    ye, yo = _upconv(h, wu[...], bu[...])
    oute[...] = ye
    outo[...] = yo


def _dec1_body(hin, w1a, b1a, w2a, b2a, wsa, bsa,
               w1b, b1b, w2b, b2b, wu, bu, oute, outo):
    h = _resblock(hin[...], w1a[...], b1a[...], w2a[...], b2a[...],
                  wsa[...], bsa[...])
    h = _resblock(h, w1b[...], b1b[...], w2b[...], b2b[...])
    ye, yo = _upconv(h, wu[...], bu[...])
    oute[...] = ye
    outo[...] = yo


def _dec0_body(hin, w1a, b1a, w2a, b2a, wsa, bsa,
               w1b, b1b, w2b, b2b, wo, bo, out):
    h = _resblock(hin[...], w1a[...], b1a[...], w2a[...], b2a[...],
                  wsa[...], bsa[...])
    h = _resblock(h, w1b[...], b1b[...], w2b[...], b2b[...])
    out[...] = _conv3(jnp.maximum(h, 0.0), wo[...], bo[...])


def _vq_gather(cb, idx):
    info = plsc.get_sparse_core_info()
    nc, ns = info.num_cores, info.num_subcores
    nw = nc * ns
    rows = idx.shape[0]
    b_per_w = rows // nw
    mesh = plsc.VectorSubcoreMesh(core_axis_name="c", subcore_axis_name="s")

    @functools.partial(
        pl.kernel, mesh=mesh,
        out_type=jax.ShapeDtypeStruct((rows, cb.shape[1]), jnp.float32),
        scratch_types=[
            pltpu.VMEM((b_per_w,), jnp.int32),
            pltpu.VMEM((b_per_w, cb.shape[1]), jnp.float32),
            pltpu.SemaphoreType.DMA,
        ],
    )
    def gather_k(table_hbm, idx_hbm, out_hbm, idx_v, rows_v, sem):
        wid = lax.axis_index("s") * nc + lax.axis_index("c")
        base = wid * b_per_w
        pltpu.sync_copy(idx_hbm.at[pl.ds(base, b_per_w)], idx_v)
        pltpu.async_copy(table_hbm.at[idx_v], rows_v, sem).wait()
        pltpu.sync_copy(rows_v, out_hbm.at[pl.ds(base, b_per_w)])

    return gather_k(cb, idx)


def _wt(p, name):
    return jnp.transpose(p[name + '_w'], (2, 1, 0)).astype(_BF)


def _res_args(p, pre, shortcut):
    a = [jnp.transpose(p[pre + '_w1'], (2, 1, 0)).astype(_BF), p[pre + '_b1'],
         jnp.transpose(p[pre + '_w2'], (2, 1, 0)).astype(_BF), p[pre + '_b2']]
    if shortcut:
        a += [p[pre + '_ws'][:, :, 0].T.astype(_BF), p[pre + '_bs']]
    return a


def _interleave(ye, yo):
    bb, tt, cc = ye.shape
    return jnp.stack([ye, yo], axis=2).reshape(bb, 2 * tt, cc)


def kernel(x, params):
    p = params
    f32 = jnp.float32

    h0, = _call(
        _enc0_body,
        [x, _wt(p, 'enc_in'), p['enc_in_b']]
        + _res_args(p, 'enc_r0_0', False)
        + _res_args(p, 'enc_r0_1', False),
        [((_B, _T, _CH), f32)])

    h1, = _call(
        _enc1_body,
        [h0[:, 0::2], h0[:, 1::2], _wt(p, 'enc_d0'), p['enc_d0_b']]
        + _res_args(p, 'enc_r1_0', True)
        + _res_args(p, 'enc_r1_1', False),
        [((_B, _T // 2, 2 * _CH), f32)])

    idx, = _call(
        _enc2_body,
        [h1[:, 0::2], h1[:, 1::2], _wt(p, 'enc_d1'), p['enc_d1_b']]
        + _res_args(p, 'enc_r2_0', True)
        + _res_args(p, 'enc_r2_1', False)
        + [_wt(p, 'enc_out'), p['enc_out_b'], p['codebook'].T],
        [((_B * _T // 4,), jnp.int32)])

    zq = jnp.take(params['codebook'], idx, axis=0)

    ye, yo = _call(
        _dec2_body,
        [zq.reshape(_B, _T // 4, _ZCH), _wt(p, 'dec_in'), p['dec_in_b']]
        + _res_args(p, 'dec_r2_0', False)
        + _res_args(p, 'dec_r2_1', False)
        + [_wt(p, 'dec_u2'), p['dec_u2_b']],
        [((_B, _T // 4, 4 * _CH), f32), ((_B, _T // 4, 4 * _CH), f32)])
    d1 = _interleave(ye, yo)

    ye, yo = _call(
        _dec1_body,
        [d1] + _res_args(p, 'dec_r1_0', True)
        + _res_args(p, 'dec_r1_1', False)
        + [_wt(p, 'dec_u1'), p['dec_u1_b']],
        [((_B, _T // 2, 2 * _CH), f32), ((_B, _T // 2, 2 * _CH), f32)])
    d0 = _interleave(ye, yo)

    y, = _call(
        _dec0_body,
        [d0] + _res_args(p, 'dec_r0_0', True)
        + _res_args(p, 'dec_r0_1', False)
        + [_wt(p, 'dec_out'), p['dec_out_b']],
        [((_B, _T, _IN_CH), f32)])

    return jnp.transpose(y, (0, 2, 1))
